# baseline (speedup 1.0000x reference)
import jax
import jax.numpy as jnp
from jax import lax
from jax.experimental import pallas as pl
from jax.experimental.pallas import tpu as pltpu
from jax.experimental.pallas import tpu_sc as plsc

OBS_DIM = 128
D_MODEL = 128
BATCH = 16384
OUT_DIM = OBS_DIM - 1 + D_MODEL

_info = plsc.get_sparse_core_info()
NC, NS, L = _info.num_cores, _info.num_subcores, _info.num_lanes
NW = NC * NS
B_PER_W = BATCH // NW
CHUNK = 128
N_CHUNKS = B_PER_W // CHUNK
N_PAIRS = N_CHUNKS // 2


def _sc_body(obs_hbm, pe_hbm, out_hbm, out_v, pe_v, idx_v,
             sem_obs0, sem_obs1, sem_g, sem_out0, sem_out1):
    wid = lax.axis_index("s") * NC + lax.axis_index("c")
    base = wid * B_PER_W
    lane = lax.iota(jnp.int32, L)
    col = jnp.full((L,), OBS_DIM - 1, dtype=jnp.int32)
    merge_cols = [lane + (OBS_DIM - 1 + L * j) for j in range(D_MODEL // L)]
    sem_obs = [sem_obs0, sem_obs1]
    sem_out = [sem_out0, sem_out1]

    def obs_copy(c, bs):
        return pltpu.make_async_copy(
            obs_hbm.at[pl.ds(base + c * CHUNK, CHUNK)],
            out_v.at[bs, :, pl.ds(0, OBS_DIM)],
            sem_obs[bs],
        )

    def out_copy(c, bs):
        return pltpu.make_async_copy(
            out_v.at[bs],
            out_hbm.at[pl.ds(base + c * CHUNK, CHUNK)],
            sem_out[bs],
        )

    obs_copy(0, 0).start()

    def pair(g, _):
        for c2 in range(2):
            bs = c2
            c = 2 * g + c2
            obs_copy(c, bs).wait()
            for i in range(CHUNK // L):
                vals = plsc.load_gather(out_v.at[bs], [lane + i * L, col])
                idx_v[bs, pl.ds(i * L, L)] = vals.astype(jnp.int32)
            h_g = pltpu.async_copy(pe_hbm.at[idx_v.at[bs]], pe_v.at[bs], sem_g)
            h_g.wait()
            if c2 == 0:
                @pl.when(g > 0)
                def _():
                    out_copy(2 * g - 1, 1).wait()
                obs_copy(c + 1, 1).start()
            else:
                @pl.when(g < N_PAIRS - 1)
                def _():
                    out_copy(c - 1, 0).wait()
                    obs_copy(c + 1, 0).start()

            @plsc.parallel_loop(0, CHUNK, unroll=2)
            def _merge(r):
                row_vec = jnp.full((L,), 0, dtype=jnp.int32) + r
                for j in range(D_MODEL // L):
                    vals = pe_v[bs, r, pl.ds(j * L, L)]
                    plsc.store_scatter(out_v.at[bs], [row_vec, merge_cols[j]],
                                       vals)

            out_copy(c, bs).start()
        return _

    lax.fori_loop(0, N_PAIRS, pair, None)
    out_copy(N_CHUNKS - 2, 0).wait()
    out_copy(N_CHUNKS - 1, 1).wait()


@jax.jit
def _run(obs, pe):
    mesh = plsc.VectorSubcoreMesh(core_axis_name="c", subcore_axis_name="s")
    return pl.kernel(
        _sc_body,
        mesh=mesh,
        compiler_params=pltpu.CompilerParams(
            needs_layout_passes=False,
            disable_bounds_checks=True,
            disable_semaphore_checks=True,
            skip_device_barrier=True,
        ),
        out_type=jax.ShapeDtypeStruct((BATCH, OUT_DIM), jnp.float32),
        scratch_types=[
            pltpu.VMEM((2, CHUNK, OUT_DIM), jnp.float32),
            pltpu.VMEM((2, CHUNK, D_MODEL), jnp.float32),
            pltpu.VMEM((2, CHUNK), jnp.int32),
            pltpu.SemaphoreType.DMA,
            pltpu.SemaphoreType.DMA,
            pltpu.SemaphoreType.DMA,
            pltpu.SemaphoreType.DMA,
            pltpu.SemaphoreType.DMA,
        ],
    )(obs, pe)


def kernel(obs, pe):
    return _run(obs, pe)


# speedup vs baseline: 1.0119x; 1.0119x over previous
import jax
import jax.numpy as jnp
from jax import lax
from jax.experimental import pallas as pl
from jax.experimental.pallas import tpu as pltpu
from jax.experimental.pallas import tpu_sc as plsc

OBS_DIM = 128
D_MODEL = 128
BATCH = 16384
OUT_DIM = OBS_DIM - 1 + D_MODEL

_info = plsc.get_sparse_core_info()
NC, NS, L = _info.num_cores, _info.num_subcores, _info.num_lanes
NW = NC * NS
B_PER_W = BATCH // NW
CHUNK = 128
N_CHUNKS = B_PER_W // CHUNK
N_PAIRS = N_CHUNKS // 2


def _sc_body(obs_hbm, pe_hbm, out_hbm, out_v, pe_v, idx_v,
             sem_obs0, sem_obs1, sem_g, sem_out0, sem_out1):
    wid = lax.axis_index("s") * NC + lax.axis_index("c")
    base = wid * B_PER_W
    lane = lax.iota(jnp.int32, L)
    col = jnp.full((L,), OBS_DIM - 1, dtype=jnp.int32)
    merge_cols = [lane + (OBS_DIM - 1 + L * j) for j in range(D_MODEL // L)]
    sem_obs = [sem_obs0, sem_obs1]
    sem_out = [sem_out0, sem_out1]

    def obs_copy(c, bs):
        return pltpu.make_async_copy(
            obs_hbm.at[pl.ds(base + c * CHUNK, CHUNK)],
            out_v.at[bs, :, pl.ds(0, OBS_DIM)],
            sem_obs[bs],
        )

    def out_copy(c, bs):
        return pltpu.make_async_copy(
            out_v.at[bs],
            out_hbm.at[pl.ds(base + c * CHUNK, CHUNK)],
            sem_out[bs],
        )

    obs_copy(0, 0).start()

    def pair(g, _):
        for c2 in range(2):
            bs = c2
            c = 2 * g + c2
            obs_copy(c, bs).wait()
            for i in range(CHUNK // L):
                vals = plsc.load_gather(out_v.at[bs], [lane + i * L, col])
                idx_v[bs, pl.ds(i * L, L)] = vals.astype(jnp.int32)
            h_g = pltpu.async_copy(pe_hbm.at[idx_v.at[bs]], pe_v.at[bs], sem_g)
            if c2 == 0:
                @pl.when(g > 0)
                def _():
                    out_copy(2 * g - 1, 1).wait()
                obs_copy(c + 1, 1).start()
            else:
                @pl.when(g < N_PAIRS - 1)
                def _():
                    out_copy(c - 1, 0).wait()
                    obs_copy(c + 1, 0).start()
            h_g.wait()

            @plsc.parallel_loop(0, CHUNK, unroll=2)
            def _merge(r):
                row_vec = jnp.full((L,), 0, dtype=jnp.int32) + r
                for j in range(D_MODEL // L):
                    vals = pe_v[bs, r, pl.ds(j * L, L)]
                    plsc.store_scatter(out_v.at[bs], [row_vec, merge_cols[j]],
                                       vals)

            out_copy(c, bs).start()
        return _

    lax.fori_loop(0, N_PAIRS, pair, None)
    out_copy(N_CHUNKS - 2, 0).wait()
    out_copy(N_CHUNKS - 1, 1).wait()


@jax.jit
def _run(obs, pe):
    mesh = plsc.VectorSubcoreMesh(core_axis_name="c", subcore_axis_name="s")
    return pl.kernel(
        _sc_body,
        mesh=mesh,
        compiler_params=pltpu.CompilerParams(
            needs_layout_passes=False,
            disable_bounds_checks=True,
            disable_semaphore_checks=True,
            skip_device_barrier=True,
        ),
        out_type=jax.ShapeDtypeStruct((BATCH, OUT_DIM), jnp.float32),
        scratch_types=[
            pltpu.VMEM((2, CHUNK, OUT_DIM), jnp.float32),
            pltpu.VMEM((2, CHUNK, D_MODEL), jnp.float32),
            pltpu.VMEM((2, CHUNK), jnp.int32),
            pltpu.SemaphoreType.DMA,
            pltpu.SemaphoreType.DMA,
            pltpu.SemaphoreType.DMA,
            pltpu.SemaphoreType.DMA,
            pltpu.SemaphoreType.DMA,
        ],
    )(obs, pe)


def kernel(obs, pe):
    return _run(obs, pe)
